# Initial kernel scaffold; baseline (speedup 1.0000x reference)
import jax
import jax.numpy as jnp
from jax.experimental import pallas as pl


def kernel(x, edge_index, edge_weight, w):
    raise NotImplementedError("write your pallas kernel here")


# speedup vs baseline: 32.1360x; 32.1360x over previous
import functools

import jax
import jax.numpy as jnp
from jax import lax
from jax.experimental import pallas as pl
from jax.experimental.pallas import tpu as pltpu
from jax.experimental.pallas import tpu_sc as plsc

N = 10000
E = 320000
D = 128
K = 2

NS = 16
CHUNK = 128
EPT = E // NS
ROWS = -(-EPT // CHUNK)
EPT_PAD = ROWS * CHUNK
NPAD = 10240
SLICE = NPAD // NS
VL = 16


def _matvec_body(x_ref, w_ref, o_ref):
    o_ref[...] = jnp.dot(x_ref[...], w_ref[...],
                         preferred_element_type=jnp.float32)


def _matvec(xp, w):
    blk = NPAD // 8
    return pl.pallas_call(
        _matvec_body,
        grid=(8,),
        in_specs=[
            pl.BlockSpec((blk, D), lambda i: (i, 0)),
            pl.BlockSpec((D, 1), lambda i: (0, 0)),
        ],
        out_specs=pl.BlockSpec((blk, 1), lambda i: (i, 0)),
        out_shape=jax.ShapeDtypeStruct((NPAD, 1), jnp.float32),
    )(xp, w)


_mesh = plsc.VectorSubcoreMesh(
    core_axis_name="c", subcore_axis_name="s", num_cores=1)


@functools.partial(
    pl.kernel,
    out_type=jax.ShapeDtypeStruct((NPAD,), jnp.float32),
    mesh=_mesh,
    compiler_params=pltpu.CompilerParams(
        use_tc_tiling_on_sc=False, needs_layout_passes=False),
    scratch_types=[
        pltpu.VMEM((NPAD,), jnp.float32),
        pltpu.VMEM((ROWS, CHUNK), jnp.int32),
        pltpu.VMEM((ROWS, CHUNK), jnp.int32),
        pltpu.VMEM((ROWS, CHUNK), jnp.float32),
        pltpu.VMEM((ROWS, CHUNK), jnp.float32),
        pltpu.VMEM((SLICE,), jnp.float32),
        pltpu.VMEM_SHARED((NPAD,), jnp.float32),
    ],
)
def _sgc_sc(y0_hbm, src_hbm, dst_hbm, ew_hbm, out_hbm,
            y_l, src_l, dst_l, ew_l, msg_l, sbuf, acc):
    sid = lax.axis_index("s")

    pltpu.sync_copy(src_hbm.at[sid], src_l)
    pltpu.sync_copy(dst_hbm.at[sid], dst_l)
    pltpu.sync_copy(ew_hbm.at[sid], ew_l)
    pltpu.sync_copy(y0_hbm, y_l)

    def _zero(i, _):
        sbuf[pl.ds(i * VL, VL)] = jnp.zeros((VL,), jnp.float32)
        return 0

    lax.fori_loop(0, SLICE // VL, _zero, 0)
    pltpu.sync_copy(sbuf, acc.at[pl.ds(sid * SLICE, SLICE)])
    plsc.subcore_barrier()

    for hop in range(K):
        def _msgs(j, _):
            for g in range(CHUNK // VL):
                s16 = src_l[j, pl.ds(g * VL, VL)]
                e16 = ew_l[j, pl.ds(g * VL, VL)]
                vals = plsc.load_gather(y_l, [s16])
                msg_l[j, pl.ds(g * VL, VL)] = vals * e16
            return 0

        lax.fori_loop(0, ROWS, _msgs, 0)

        def _scat(j, _):
            pltpu.sync_copy(msg_l.at[j], acc.at[dst_l.at[j]], add=True)
            return 0

        lax.fori_loop(0, ROWS, _scat, 0)
        plsc.subcore_barrier()
        if hop + 1 < K:
            pltpu.sync_copy(acc, y_l)
            plsc.subcore_barrier()
            lax.fori_loop(0, SLICE // VL, _zero, 0)
            pltpu.sync_copy(sbuf, acc.at[pl.ds(sid * SLICE, SLICE)])
            plsc.subcore_barrier()

    pltpu.sync_copy(acc.at[pl.ds(sid * SLICE, SLICE)], sbuf)

    def _sig(i, _):
        z = sbuf[pl.ds(i * VL, VL)]
        sbuf[pl.ds(i * VL, VL)] = 1.0 / (1.0 + jnp.exp(-z))
        return 0

    lax.fori_loop(0, SLICE // VL, _sig, 0)
    pltpu.sync_copy(sbuf, out_hbm.at[pl.ds(sid * SLICE, SLICE)])


def kernel(x, edge_index, edge_weight, w):
    src = edge_index[0]
    dst = edge_index[1]
    pad = NS * EPT_PAD - E
    srcp = jnp.concatenate(
        [src, jnp.zeros((pad,), jnp.int32)]).reshape(NS, ROWS, CHUNK)
    dstp = jnp.concatenate(
        [dst, jnp.full((pad,), N, jnp.int32)]).reshape(NS, ROWS, CHUNK)
    ewp = jnp.concatenate(
        [edge_weight, jnp.zeros((pad,), jnp.float32)]).reshape(NS, ROWS, CHUNK)

    xp = jnp.pad(x, ((0, NPAD - N), (0, 0)))
    y0 = _matvec(xp, w).reshape(NPAD)

    res = _sgc_sc(y0, srcp, dstp, ewp)
    return res[:N].reshape(N, 1)


# speedup vs baseline: 40.7548x; 1.2682x over previous
import functools

import jax
import jax.numpy as jnp
from jax import lax
from jax.experimental import pallas as pl
from jax.experimental.pallas import tpu as pltpu
from jax.experimental.pallas import tpu_sc as plsc

N = 10000
E = 320000
D = 128
K = 2

NS = 16
CHUNK = 128
EROWS = E // CHUNK
RPT = 157
RLAST = EROWS - 15 * RPT
NPAD = 10240
SLICE = NPAD // NS
VL = 16
LAG = 12


def _matvec_body(x_ref, w_ref, o_ref):
    o_ref[...] = jnp.dot(x_ref[...], w_ref[...],
                         preferred_element_type=jnp.float32)


def _matvec(x, w):
    blk = N // 10
    return pl.pallas_call(
        _matvec_body,
        grid=(10,),
        in_specs=[
            pl.BlockSpec((blk, D), lambda i: (i, 0)),
            pl.BlockSpec((D, 1), lambda i: (0, 0)),
        ],
        out_specs=pl.BlockSpec((blk, 1), lambda i: (i, 0)),
        out_shape=jax.ShapeDtypeStruct((N, 1), jnp.float32),
    )(x, w)


_mesh = plsc.VectorSubcoreMesh(
    core_axis_name="c", subcore_axis_name="s", num_cores=1)


@functools.partial(
    pl.kernel,
    out_type=jax.ShapeDtypeStruct((NPAD,), jnp.float32),
    mesh=_mesh,
    compiler_params=pltpu.CompilerParams(
        use_tc_tiling_on_sc=False, needs_layout_passes=False),
    scratch_types=[
        pltpu.VMEM((NPAD,), jnp.float32),
        pltpu.VMEM((RPT, CHUNK), jnp.int32),
        pltpu.VMEM((RPT, CHUNK), jnp.int32),
        pltpu.VMEM((RPT, CHUNK), jnp.float32),
        pltpu.VMEM((RPT, CHUNK), jnp.float32),
        pltpu.VMEM((SLICE,), jnp.float32),
        pltpu.VMEM_SHARED((NPAD,), jnp.float32),
        pltpu.SemaphoreType.DMA,
    ],
)
def _sgc_sc(y0_hbm, src_hbm, dst_hbm, ew_hbm, out_hbm,
            y_l, src_l, dst_l, ew_l, msg_l, sbuf, acc, sem):
    sid = lax.axis_index("s")
    nrows = jnp.where(sid == NS - 1, RLAST, RPT)
    row0 = sid * RPT

    @pl.when(sid < NS - 1)
    def _():
        pltpu.sync_copy(src_hbm.at[pl.ds(row0, RPT)], src_l)
        pltpu.sync_copy(dst_hbm.at[pl.ds(row0, RPT)], dst_l)
        pltpu.sync_copy(ew_hbm.at[pl.ds(row0, RPT)], ew_l)

    @pl.when(sid == NS - 1)
    def _():
        pltpu.sync_copy(src_hbm.at[pl.ds(row0, RLAST)],
                        src_l.at[pl.ds(0, RLAST)])
        pltpu.sync_copy(dst_hbm.at[pl.ds(row0, RLAST)],
                        dst_l.at[pl.ds(0, RLAST)])
        pltpu.sync_copy(ew_hbm.at[pl.ds(row0, RLAST)],
                        ew_l.at[pl.ds(0, RLAST)])

    pltpu.sync_copy(y0_hbm, y_l.at[pl.ds(0, N)])

    def _zero(i, _):
        sbuf[pl.ds(i * VL, VL)] = jnp.zeros((VL,), jnp.float32)
        return 0

    lax.fori_loop(0, SLICE // VL, _zero, 0)
    pltpu.sync_copy(sbuf, acc.at[pl.ds(sid * SLICE, SLICE)])
    plsc.subcore_barrier()

    def _scat_wait():
        pltpu.make_async_copy(
            msg_l.at[0], acc.at[dst_l.at[0]], sem).wait()

    for hop in range(K):
        def _row(j, _):
            for g in range(CHUNK // VL):
                s16 = src_l[j, pl.ds(g * VL, VL)]
                e16 = ew_l[j, pl.ds(g * VL, VL)]
                vals = plsc.load_gather(y_l, [s16])
                msg_l[j, pl.ds(g * VL, VL)] = vals * e16
            pltpu.async_copy(msg_l.at[j], acc.at[dst_l.at[j]], sem, add=True)

            @pl.when(j >= LAG)
            def _():
                _scat_wait()

            return 0

        lax.fori_loop(0, nrows, _row, 0)

        def _drain(j, _):
            _scat_wait()
            return 0

        lax.fori_loop(0, LAG, _drain, 0)
        plsc.subcore_barrier()
        if hop + 1 < K:
            pltpu.sync_copy(acc, y_l)
            plsc.subcore_barrier()
            lax.fori_loop(0, SLICE // VL, _zero, 0)
            pltpu.sync_copy(sbuf, acc.at[pl.ds(sid * SLICE, SLICE)])
            plsc.subcore_barrier()

    pltpu.sync_copy(acc.at[pl.ds(sid * SLICE, SLICE)], sbuf)

    def _sig(i, _):
        z = sbuf[pl.ds(i * VL, VL)]
        sbuf[pl.ds(i * VL, VL)] = 1.0 / (1.0 + jnp.exp(-z))
        return 0

    lax.fori_loop(0, SLICE // VL, _sig, 0)
    pltpu.sync_copy(sbuf, out_hbm.at[pl.ds(sid * SLICE, SLICE)])


def kernel(x, edge_index, edge_weight, w):
    src2d = edge_index[0].reshape(EROWS, CHUNK)
    dst2d = edge_index[1].reshape(EROWS, CHUNK)
    ew2d = edge_weight.reshape(EROWS, CHUNK)
    y0 = _matvec(x, w).reshape(N)
    res = _sgc_sc(y0, src2d, dst2d, ew2d)
    return res[:N].reshape(N, 1)


# speedup vs baseline: 67.3863x; 1.6535x over previous
import functools

import jax
import jax.numpy as jnp
from jax import lax
from jax.experimental import pallas as pl
from jax.experimental.pallas import tpu as pltpu
from jax.experimental.pallas import tpu_sc as plsc

N = 10000
E = 320000
D = 128
K = 2

NS = 16
CHUNK = 128
EROWS = E // CHUNK
RPT = 157
RLAST = EROWS - 15 * RPT
EPT_PAD = RPT * CHUNK
NPAD = 10240
SLICE = NPAD // NS
VL = 16
SCHUNKS = [(0, 20), (20, 20), (40, 20), (60, 20),
           (80, 20), (100, 20), (120, 20), (140, 17)]


def _matvec_body(x_ref, w_ref, o_ref):
    o_ref[...] = jax.lax.dot_general(
        w_ref[...], x_ref[...],
        dimension_numbers=(((0,), (1,)), ((), ())),
        preferred_element_type=jnp.float32).reshape(N)


def _matvec(x, w):
    return pl.pallas_call(
        _matvec_body,
        in_specs=[
            pl.BlockSpec((N, D), lambda: (0, 0)),
            pl.BlockSpec((D, 1), lambda: (0, 0)),
        ],
        out_specs=pl.BlockSpec((N,), lambda: (0,)),
        out_shape=jax.ShapeDtypeStruct((N,), jnp.float32),
    )(x, w)


_mesh = plsc.VectorSubcoreMesh(
    core_axis_name="c", subcore_axis_name="s", num_cores=1)


@functools.partial(
    pl.kernel,
    out_type=jax.ShapeDtypeStruct((NPAD,), jnp.float32),
    mesh=_mesh,
    compiler_params=pltpu.CompilerParams(
        use_tc_tiling_on_sc=False, needs_layout_passes=False),
    scratch_types=[
        pltpu.VMEM((NPAD,), jnp.float32),
        pltpu.VMEM((RPT, CHUNK), jnp.int32),
        pltpu.VMEM((RPT, CHUNK), jnp.int32),
        pltpu.VMEM((RPT, CHUNK), jnp.float32),
        pltpu.VMEM((EPT_PAD,), jnp.float32),
        pltpu.VMEM((EPT_PAD,), jnp.int32),
        pltpu.VMEM((SLICE,), jnp.float32),
        pltpu.VMEM_SHARED((NPAD,), jnp.float32),
        pltpu.VMEM_SHARED((NPAD,), jnp.float32),
        pltpu.SemaphoreType.DMA,
        pltpu.SemaphoreType.DMA,
    ],
)
def _sgc_sc(y0_hbm, ei_hbm, ew_hbm, out_hbm,
            y_l, src_l, dst_l, ew_l, msg_f, dst_f, sbuf,
            acc0, acc1, ssem, sem):
    sid = lax.axis_index("s")
    row0 = sid * RPT

    @pl.when(sid < NS - 1)
    def _():
        pltpu.async_copy(ei_hbm.at[pl.ds(row0, RPT), 0], src_l, ssem)
        pltpu.async_copy(ei_hbm.at[pl.ds(row0, RPT), 1], dst_l, ssem)
        pltpu.async_copy(ew_hbm.at[pl.ds(row0, RPT)], ew_l, ssem)

    @pl.when(sid == NS - 1)
    def _():
        pltpu.async_copy(ei_hbm.at[pl.ds(row0, RLAST), 0],
                         src_l.at[pl.ds(0, RLAST)], ssem)
        pltpu.async_copy(ei_hbm.at[pl.ds(row0, RLAST), 1],
                         dst_l.at[pl.ds(0, RLAST)], ssem)
        pltpu.async_copy(ew_hbm.at[pl.ds(row0, RLAST)],
                         ew_l.at[pl.ds(0, RLAST)], ssem)
    pltpu.async_copy(y0_hbm, y_l.at[pl.ds(0, N)], ssem)

    def _zero(i, _):
        sbuf[pl.ds(i * VL, VL)] = jnp.zeros((VL,), jnp.float32)
        return 0

    lax.fori_loop(0, SLICE // VL, _zero, 0)

    @pl.when(sid < NS - 1)
    def _():
        pltpu.make_async_copy(ei_hbm.at[pl.ds(0, RPT), 0], src_l, ssem).wait()
        pltpu.make_async_copy(ei_hbm.at[pl.ds(0, RPT), 1], dst_l, ssem).wait()
        pltpu.make_async_copy(ew_hbm.at[pl.ds(0, RPT)], ew_l, ssem).wait()

    @pl.when(sid == NS - 1)
    def _():
        pltpu.make_async_copy(ei_hbm.at[pl.ds(0, RLAST), 0],
                              src_l.at[pl.ds(0, RLAST)], ssem).wait()
        pltpu.make_async_copy(ei_hbm.at[pl.ds(0, RLAST), 1],
                              dst_l.at[pl.ds(0, RLAST)], ssem).wait()
        pltpu.make_async_copy(ew_hbm.at[pl.ds(0, RLAST)],
                              ew_l.at[pl.ds(0, RLAST)], ssem).wait()
    pltpu.make_async_copy(y0_hbm, y_l.at[pl.ds(0, N)], ssem).wait()

    @pl.when(sid == NS - 1)
    def _():
        def _fill(j, _):
            for g in range(CHUNK // VL):
                src_l[j, pl.ds(g * VL, VL)] = jnp.zeros((VL,), jnp.int32)
                ew_l[j, pl.ds(g * VL, VL)] = jnp.zeros((VL,), jnp.float32)
                dst_l[j, pl.ds(g * VL, VL)] = jnp.full((VL,), N, jnp.int32)
            return 0
        lax.fori_loop(RLAST, RPT, _fill, 0)

    pltpu.sync_copy(sbuf, acc0.at[pl.ds(sid * SLICE, SLICE)])
    pltpu.sync_copy(sbuf, acc1.at[pl.ds(sid * SLICE, SLICE)])
    plsc.subcore_barrier()

    for hop in range(K):
        acc = acc0 if hop == 0 else acc1

        def _row(j, _):
            for g in range(CHUNK // VL):
                s16 = src_l[j, pl.ds(g * VL, VL)]
                e16 = ew_l[j, pl.ds(g * VL, VL)]
                vals = plsc.load_gather(y_l, [s16])
                msg_f[pl.ds(j * CHUNK + g * VL, VL)] = vals * e16
                if hop == 0:
                    dst_f[pl.ds(j * CHUNK + g * VL, VL)] = \
                        dst_l[j, pl.ds(g * VL, VL)]
            return 0

        for (r0, rn) in SCHUNKS:
            lax.fori_loop(r0, r0 + rn, _row, 0)
            pltpu.async_copy(
                msg_f.at[pl.ds(r0 * CHUNK, rn * CHUNK)],
                acc.at[dst_f.at[pl.ds(r0 * CHUNK, rn * CHUNK)]],
                sem, add=True)
        for (r0, rn) in SCHUNKS:
            pltpu.make_async_copy(
                msg_f.at[pl.ds(r0 * CHUNK, rn * CHUNK)],
                acc.at[dst_f.at[pl.ds(r0 * CHUNK, rn * CHUNK)]],
                sem).wait()
        plsc.subcore_barrier()
        if hop + 1 < K:
            pltpu.sync_copy(acc.at[pl.ds(sid * SLICE, SLICE)],
                            out_hbm.at[pl.ds(sid * SLICE, SLICE)])
            plsc.subcore_barrier()
            pltpu.sync_copy(out_hbm, y_l)

    pltpu.sync_copy(acc1.at[pl.ds(sid * SLICE, SLICE)], sbuf)

    def _sig(i, _):
        z = sbuf[pl.ds(i * VL, VL)]
        sbuf[pl.ds(i * VL, VL)] = 1.0 / (1.0 + jnp.exp(-z))
        return 0

    lax.fori_loop(0, SLICE // VL, _sig, 0)
    pltpu.sync_copy(sbuf, out_hbm.at[pl.ds(sid * SLICE, SLICE)])


def kernel(x, edge_index, edge_weight, w):
    ei3 = edge_index.reshape(2, EROWS, CHUNK).transpose(1, 0, 2)
    ew2d = edge_weight.reshape(EROWS, CHUNK)
    y0 = _matvec(x, w)
    res = _sgc_sc(y0, ei3, ew2d)
    return res[:N].reshape(N, 1)
